# Initial kernel scaffold; baseline (speedup 1.0000x reference)
import jax
import jax.numpy as jnp
from jax.experimental import pallas as pl


def kernel(queries, items):
    raise NotImplementedError("write your pallas kernel here")


# speedup vs baseline: 9.2925x; 9.2925x over previous
import jax
import jax.numpy as jnp
from jax.experimental import pallas as pl


def _body(q_ref, items_ref, out_ref):
    step = pl.program_id(0)

    @pl.when(step == 0)
    def _init():
        out_ref[...] = jnp.zeros_like(out_ref)

    tb = q_ref.shape[0]
    n_vars = q_ref.shape[1]
    n_mem = items_ref.shape[1]
    acc = jnp.float32(0.0)
    for v in range(n_vars):
        q = q_ref[:, v, :]
        it = items_ref[v]
        s = jax.lax.dot_general(
            q, it, (((1,), (1,)), ((), ())),
            preferred_element_type=jnp.float32)
        rowmax = jnp.max(s, axis=1, keepdims=True)
        iota = jax.lax.broadcasted_iota(jnp.int32, (tb, n_mem), 1)
        idx = jnp.min(jnp.where(s == rowmax, iota, n_mem), axis=1,
                      keepdims=True)
        nsq = jnp.sum(it * it, axis=1)[None, :]
        nsel = jnp.sum(jnp.where(iota == idx, nsq, 0.0))
        acc += jnp.sum(q * q) - 2.0 * jnp.sum(rowmax) + nsel
    out_ref[...] += jnp.full((8, 128), acc * (1.0 / 1024.0), jnp.float32)


def kernel(queries, items):
    t, n_vars, c = queries.shape
    n_mem = items.shape[1]
    tb = 512
    nt = t // tb
    part = pl.pallas_call(
        _body,
        grid=(nt,),
        in_specs=[
            pl.BlockSpec((tb, n_vars, c), lambda i: (i, 0, 0)),
            pl.BlockSpec((n_vars, n_mem, c), lambda i: (0, 0, 0)),
        ],
        out_specs=pl.BlockSpec((8, 128), lambda i: (0, 0)),
        out_shape=jax.ShapeDtypeStruct((8, 128), jnp.float32),
    )(queries, items)
    return jnp.sum(part) / (t * n_vars * c)


# speedup vs baseline: 10.0277x; 1.0791x over previous
import jax
import jax.numpy as jnp
from jax.experimental import pallas as pl


def _body(q_ref, items_ref, out_ref):
    step = pl.program_id(0)

    @pl.when(step == 0)
    def _init():
        out_ref[...] = jnp.zeros_like(out_ref)

    n_vars = q_ref.shape[1]
    n_mem = items_ref.shape[1]
    lane = jax.lax.broadcasted_iota(jnp.int32, (1, n_mem), 1)
    code = (n_mem - 1) - lane
    acc = jnp.float32(0.0)
    for v in range(n_vars):
        q = q_ref[:, v, :]
        it = items_ref[v]
        s = jax.lax.dot_general(
            q.astype(jnp.bfloat16), it.astype(jnp.bfloat16),
            (((1,), (1,)), ((), ())),
            preferred_element_type=jnp.float32)
        b = jax.lax.bitcast_convert_type(s, jnp.int32)
        s_emb = jax.lax.bitcast_convert_type((b & (-128)) | code,
                                             jnp.float32)
        rowmax = jnp.max(s_emb, axis=1, keepdims=True)
        onehot = s_emb == rowmax
        nsq = jnp.sum(it * it, axis=1)[None, :]
        nsel = jnp.sum(jnp.where(onehot, nsq, 0.0))
        acc += jnp.sum(q * q) - 2.0 * jnp.sum(rowmax) + nsel
    out_ref[...] += jnp.full((8, 128), acc * (1.0 / 1024.0), jnp.float32)


def kernel(queries, items):
    t, n_vars, c = queries.shape
    n_mem = items.shape[1]
    tb = 512
    nt = t // tb
    part = pl.pallas_call(
        _body,
        grid=(nt,),
        in_specs=[
            pl.BlockSpec((tb, n_vars, c), lambda i: (i, 0, 0)),
            pl.BlockSpec((n_vars, n_mem, c), lambda i: (0, 0, 0)),
        ],
        out_specs=pl.BlockSpec((8, 128), lambda i: (0, 0)),
        out_shape=jax.ShapeDtypeStruct((8, 128), jnp.float32),
    )(queries, items)
    return jnp.sum(part) / (t * n_vars * c)
